# baseline (speedup 1.0000x reference)
import functools

import jax
import jax.numpy as jnp
from jax import lax
from jax.experimental import pallas as pl
from jax.experimental.pallas import tpu as pltpu
from jax.experimental.pallas import tpu_sc as plsc

_B, _T, _C = 4, 8192, 768
_TS = 2048
_NC, _NS = 2, 16
_NW = _NC * _NS
_TPW = _TS // _NW
_R = 32
_NTILES = _TPW // _R
_LANES = _C // 16
_BT = 2048
_BTM = 1024


def _sc_body(x_hbm, pe_hbm, out_hbm, xb0, xb1, xb2, pb0, pb1,
             sx0, sx1, sx2, so0, so1, so2, sp0, sp1):
    wid = lax.axis_index("s") * _NC + lax.axis_index("c")
    t0 = wid * _TPW
    xbufs, pbufs = (xb0, xb1, xb2), (pb0, pb1)
    sxs, sos, sps = (sx0, sx1, sx2), (so0, so1, so2), (sp0, sp1)

    chunks = [(tile, b) for tile in range(_NTILES) for b in range(_B)]
    n = len(chunks)

    def x_row(k):
        tile, b = chunks[k]
        return b * _T + t0 + tile * _R

    def out_row(k):
        tile, b = chunks[k]
        return b * _TS + t0 + tile * _R

    pe_in = [None] * _NTILES
    x_in = [None] * n
    out_dma = [None] * n
    pe_in[0] = pltpu.async_copy(
        pe_hbm.at[pl.ds(t0, _R), :], pbufs[0], sps[0])
    x_in[0] = pltpu.async_copy(
        x_hbm.at[pl.ds(x_row(0), _R), :], xbufs[0], sxs[0])

    for k in range(n):
        tile, b = chunks[k]
        slot = k % 3
        xbuf = xbufs[slot]
        if k + 1 < n:
            if k >= 2 and out_dma[k - 2] is not None:
                out_dma[k - 2].wait()
                out_dma[k - 2] = None
            x_in[k + 1] = pltpu.async_copy(
                x_hbm.at[pl.ds(x_row(k + 1), _R), :],
                xbufs[(k + 1) % 3], sxs[(k + 1) % 3])
        if b == 0 and tile + 1 < _NTILES:
            nt = tile + 1
            pe_in[nt] = pltpu.async_copy(
                pe_hbm.at[pl.ds(t0 + nt * _R, _R), :],
                pbufs[nt % 2], sps[nt % 2])
        x_in[k].wait()
        if b == 0:
            pe_in[tile].wait()
        pbuf = pbufs[tile % 2]

        def _add_row(r, carry):
            for c in range(_LANES):
                sl = pl.ds(c * 16, 16)
                xbuf[r, sl] = xbuf[r, sl] + pbuf[r, sl]
            return carry

        lax.fori_loop(0, _R, _add_row, 0)
        out_dma[k] = pltpu.async_copy(
            xbuf, out_hbm.at[pl.ds(out_row(k), _R), :], sos[slot])

    for k in range(max(0, n - 3), n):
        if out_dma[k] is not None:
            out_dma[k].wait()


def _tc_body(x_ref, pe_ref, o_ref):
    o_ref[...] = x_ref[...] + pe_ref[...][None, :, :]


def _merge_body(full_ref, sc_ref, o_ref):
    o_ref[...] = sc_ref[...]


def kernel(x, pe_table):
    B, T, C = x.shape
    nt = (T - _TS) // _BT
    toff = _TS // _BT
    tc_full = pl.pallas_call(
        _tc_body,
        grid=(nt, B),
        in_specs=[
            pl.BlockSpec((1, _BT, C), lambda t, b: (b, t + toff, 0)),
            pl.BlockSpec((_BT, C), lambda t, b: (t + toff, 0)),
        ],
        out_specs=pl.BlockSpec((1, _BT, C), lambda t, b: (b, t + toff, 0)),
        out_shape=jax.ShapeDtypeStruct((B, T, C), x.dtype),
    )(x, pe_table[:T])

    sc_add = functools.partial(
        pl.kernel,
        mesh=plsc.VectorSubcoreMesh(core_axis_name="c", subcore_axis_name="s"),
        out_type=jax.ShapeDtypeStruct((B * _TS, C), jnp.float32),
        scratch_types=[
            pltpu.VMEM((_R, C), jnp.float32),
            pltpu.VMEM((_R, C), jnp.float32),
            pltpu.VMEM((_R, C), jnp.float32),
            pltpu.VMEM((_R, C), jnp.float32),
            pltpu.VMEM((_R, C), jnp.float32),
            pltpu.SemaphoreType.DMA,
            pltpu.SemaphoreType.DMA,
            pltpu.SemaphoreType.DMA,
            pltpu.SemaphoreType.DMA,
            pltpu.SemaphoreType.DMA,
            pltpu.SemaphoreType.DMA,
            pltpu.SemaphoreType.DMA,
            pltpu.SemaphoreType.DMA,
        ],
    )(_sc_body)
    sc_out = sc_add(x.reshape(B * T, C), pe_table[:_TS]).reshape(B, _TS, C)

    return pl.pallas_call(
        _merge_body,
        grid=(_TS // _BTM, B),
        in_specs=[
            pl.BlockSpec((1, _BTM, C), lambda t, b: (b, t, 0)),
            pl.BlockSpec((1, _BTM, C), lambda t, b: (b, t, 0)),
        ],
        out_specs=pl.BlockSpec((1, _BTM, C), lambda t, b: (b, t, 0)),
        out_shape=jax.ShapeDtypeStruct((B, T, C), x.dtype),
        input_output_aliases={0: 0},
    )(tc_full, sc_out)


# speedup vs baseline: 1.6516x; 1.6516x over previous
import jax
import jax.numpy as jnp
from jax.experimental import pallas as pl


_BT = 2048


def _add_pe_kernel(x_ref, pe_ref, o_ref):
    o_ref[...] = x_ref[...] + pe_ref[...][None, :, :]


def kernel(x, pe_table):
    B, T, C = x.shape
    grid = (T // _BT, B)
    return pl.pallas_call(
        _add_pe_kernel,
        grid=grid,
        in_specs=[
            pl.BlockSpec((1, _BT, C), lambda t, b: (b, t, 0)),
            pl.BlockSpec((_BT, C), lambda t, b: (t, 0)),
        ],
        out_specs=pl.BlockSpec((1, _BT, C), lambda t, b: (b, t, 0)),
        out_shape=jax.ShapeDtypeStruct((B, T, C), x.dtype),
    )(x, pe_table[:T])


# speedup vs baseline: 1.6736x; 1.0133x over previous
import jax
import jax.numpy as jnp
from jax.experimental import pallas as pl


_BT = 1024


def _add_pe_kernel(x_ref, pe_ref, o_ref):
    o_ref[...] = x_ref[...] + pe_ref[...][None, :, :]


def kernel(x, pe_table):
    B, T, C = x.shape
    grid = (T // _BT,)
    return pl.pallas_call(
        _add_pe_kernel,
        grid=grid,
        in_specs=[
            pl.BlockSpec((4, _BT, C), lambda t: (0, t, 0)),
            pl.BlockSpec((_BT, C), lambda t: (t, 0)),
        ],
        out_specs=pl.BlockSpec((4, _BT, C), lambda t: (0, t, 0)),
        out_shape=jax.ShapeDtypeStruct((B, T, C), x.dtype),
    )(x, pe_table[:T])
